# Initial kernel scaffold; baseline (speedup 1.0000x reference)
import jax
import jax.numpy as jnp
from jax.experimental import pallas as pl


def kernel(x_patient, x_lab, x_disease, src_pl, dst_pl, src_pd, dst_pd, W_pat, b_pat, W_lab, b_lab, W_dis, b_dis, c1_pl_Wl, c1_pl_bl, c1_pl_Wr, c1_pd_Wl, c1_pd_bl, c1_pd_Wr, c1_lp_Wl, c1_lp_bl, c1_lp_Wr, c1_dp_Wl, c1_dp_bl, c1_dp_Wr, c2_pl_Wl, c2_pl_bl, c2_pl_Wr, c2_pd_Wl, c2_pd_bl, c2_pd_Wr, c2_lp_Wl, c2_lp_bl, c2_lp_Wr, c2_dp_Wl, c2_dp_bl, c2_dp_Wr, bn1_p_g, bn1_p_b, bn1_l_g, bn1_l_b, bn1_d_g, bn1_d_b, bn2_p_g, bn2_p_b, bn2_l_g, bn2_l_b, bn2_d_g, bn2_d_b, W_fc1, b_fc1, W_fc2, b_fc2):
    raise NotImplementedError("write your pallas kernel here")


# speedup vs baseline: 5.0313x; 5.0313x over previous
import functools
import math

import jax
import jax.numpy as jnp
from jax import lax
from jax.experimental import pallas as pl
from jax.experimental.pallas import tpu as pltpu
from jax.experimental.pallas import tpu_sc as plsc

NP_N, NL_N, ND_N = 50000, 5000, 500
EPL_PAD, EPD_PAD = 819200, 229376
H = 64
OUT = 50
LAB_PAD, DIS_PAD, PAT_PAD = 5120, 512, 50048
DB = 16
DC = 8
CHUNK = 1024
KROWS = CHUNK // 128

_mesh = plsc.VectorSubcoreMesh(core_axis_name="c", subcore_axis_name="s")
_SC_PARAMS = pltpu.CompilerParams(use_tc_tiling_on_sc=False)

f32 = jnp.float32
i32 = jnp.int32


def _edge_phase(idx_g_hbm, idx_s_hbm, row_base, n_chunks, gi, si, rows, acc,
                table_ref, sem_g, sem_s):
    def body(t, carry):
        rb = row_base + t * KROWS
        pltpu.sync_copy(idx_g_hbm.at[pl.ds(rb, KROWS)], gi)
        pltpu.sync_copy(idx_s_hbm.at[pl.ds(rb, KROWS)], si)
        gets = []
        for j in range(KROWS):
            gets.append(pltpu.async_copy(
                table_ref.at[gi.at[j]], rows.at[pl.ds(j * 128, 128)], sem_g))
        for d in gets:
            d.wait()
        puts = []
        for j in range(KROWS):
            puts.append(pltpu.async_copy(
                rows.at[pl.ds(j * 128, 128)], acc.at[si.at[j]], sem_s,
                add=True))
        for d in puts:
            d.wait()
        return carry

    lax.fori_loop(0, n_chunks, body, 0)


def _count_phase(idx_s_hbm, row_base, n_chunks, si, ones_rows, acc, sem_s):
    def body(t, carry):
        rb = row_base + t * KROWS
        pltpu.sync_copy(idx_s_hbm.at[pl.ds(rb, KROWS)], si)
        puts = []
        for j in range(KROWS):
            puts.append(pltpu.async_copy(
                ones_rows.at[pl.ds(j * 128, 128)], acc.at[si.at[j]], sem_s,
                add=True))
        for d in puts:
            d.wait()
        return carry

    lax.fori_loop(0, n_chunks, body, 0)


def _make_agg_ld(width, with_counts):
    out_type = [
        jax.ShapeDtypeStruct((2, LAB_PAD, width), f32),
        jax.ShapeDtypeStruct((2, DIS_PAD, width), f32),
    ]
    scratch = [
        pltpu.VMEM((KROWS, 128), i32),
        pltpu.VMEM((KROWS, 128), i32),
        pltpu.VMEM((CHUNK, width), f32),
        pltpu.VMEM_SHARED((LAB_PAD, width), f32),
        pltpu.VMEM_SHARED((DIS_PAD, width), f32),
        pltpu.SemaphoreType.DMA,
        pltpu.SemaphoreType.DMA,
    ]
    if with_counts:
        out_type += [
            jax.ShapeDtypeStruct((2, LAB_PAD, DC), f32),
            jax.ShapeDtypeStruct((2, DIS_PAD, DC), f32),
        ]
        scratch += [
            pltpu.VMEM((CHUNK, DC), f32),
            pltpu.VMEM_SHARED((LAB_PAD, DC), f32),
            pltpu.VMEM_SHARED((DIS_PAD, DC), f32),
        ]

    rows_per_tile_pl = EPL_PAD // 32 // 128
    rows_per_tile_pd = EPD_PAD // 32 // 128

    def body(*refs):
        if with_counts:
            (tp, spl_g, dpl_s, spd_g, dpd_s, zl, zd, zcl, zcd, ones_hbm,
             out_l, out_d, out_cl, out_cd,
             gi, si, rows, acc_l, acc_d, sem_g, sem_s,
             ones_rows, acc_cl, acc_cd) = refs
        else:
            (tp, spl_g, dpl_s, spd_g, dpd_s, zl, zd, out_l, out_d,
             gi, si, rows, acc_l, acc_d, sem_g, sem_s) = refs
        core = lax.axis_index("c")
        sid = lax.axis_index("s")
        tile = core * 16 + sid
        nl = LAB_PAD // 16
        nd = DIS_PAD // 16
        pltpu.sync_copy(zl.at[pl.ds(sid * nl, nl)],
                        acc_l.at[pl.ds(sid * nl, nl)])
        pltpu.sync_copy(zd.at[pl.ds(sid * nd, nd)],
                        acc_d.at[pl.ds(sid * nd, nd)])
        if with_counts:
            pltpu.sync_copy(zcl.at[pl.ds(sid * nl, nl)],
                            acc_cl.at[pl.ds(sid * nl, nl)])
            pltpu.sync_copy(zcd.at[pl.ds(sid * nd, nd)],
                            acc_cd.at[pl.ds(sid * nd, nd)])
            pltpu.sync_copy(ones_hbm, ones_rows)
        plsc.subcore_barrier()
        _edge_phase(spl_g, dpl_s, tile * rows_per_tile_pl,
                    rows_per_tile_pl // KROWS, gi, si, rows, acc_l, tp,
                    sem_g, sem_s)
        _edge_phase(spd_g, dpd_s, tile * rows_per_tile_pd,
                    rows_per_tile_pd // KROWS, gi, si, rows, acc_d, tp,
                    sem_g, sem_s)
        if with_counts:
            _count_phase(dpl_s, tile * rows_per_tile_pl,
                         rows_per_tile_pl // KROWS, si, ones_rows, acc_cl,
                         sem_s)
            _count_phase(dpd_s, tile * rows_per_tile_pd,
                         rows_per_tile_pd // KROWS, si, ones_rows, acc_cd,
                         sem_s)
        plsc.subcore_barrier()
        pltpu.sync_copy(acc_l.at[pl.ds(sid * nl, nl)],
                        out_l.at[core, pl.ds(sid * nl, nl)])
        pltpu.sync_copy(acc_d.at[pl.ds(sid * nd, nd)],
                        out_d.at[core, pl.ds(sid * nd, nd)])
        if with_counts:
            pltpu.sync_copy(acc_cl.at[pl.ds(sid * nl, nl)],
                            out_cl.at[core, pl.ds(sid * nl, nl)])
            pltpu.sync_copy(acc_cd.at[pl.ds(sid * nd, nd)],
                            out_cd.at[core, pl.ds(sid * nd, nd)])

    return pl.kernel(body, out_type=out_type, mesh=_mesh,
                     scratch_types=scratch, compiler_params=_SC_PARAMS)


def _make_agg_pat_small(n_edges_pad):
    out_type = jax.ShapeDtypeStruct((2, PAT_PAD, DC), f32)
    scratch = [
        pltpu.VMEM((KROWS, 128), i32),
        pltpu.VMEM((KROWS, 128), i32),
        pltpu.VMEM((CHUNK, DC), f32),
        pltpu.VMEM_SHARED((PAT_PAD, DC), f32),
        pltpu.SemaphoreType.DMA,
        pltpu.SemaphoreType.DMA,
    ]
    rows_per_tile = n_edges_pad // 32 // 128

    def body(t, idx_g, idx_s, zp, out, gi, si, rows, acc, sem_g, sem_s):
        core = lax.axis_index("c")
        sid = lax.axis_index("s")
        tile = core * 16 + sid
        npr = PAT_PAD // 16
        pltpu.sync_copy(zp.at[pl.ds(sid * npr, npr)],
                        acc.at[pl.ds(sid * npr, npr)])
        plsc.subcore_barrier()
        _edge_phase(idx_g, idx_s, tile * rows_per_tile,
                    rows_per_tile // KROWS, gi, si, rows, acc, t,
                    sem_g, sem_s)
        plsc.subcore_barrier()
        pltpu.sync_copy(acc.at[pl.ds(sid * npr, npr)],
                        out.at[core, pl.ds(sid * npr, npr)])

    return pl.kernel(body, out_type=out_type, mesh=_mesh,
                     scratch_types=scratch, compiler_params=_SC_PARAMS)


def _make_agg_pat(n_edges_pad):
    out_type = jax.ShapeDtypeStruct((4, PAT_PAD, DB), f32)
    scratch = [
        pltpu.VMEM((KROWS, 128), i32),
        pltpu.VMEM((KROWS, 128), i32),
        pltpu.VMEM((CHUNK, DB), f32),
        pltpu.VMEM_SHARED((PAT_PAD, DB), f32),
        pltpu.SemaphoreType.DMA,
        pltpu.SemaphoreType.DMA,
    ]
    rows_per_sub = n_edges_pad // 16 // 128
    n_chunks = rows_per_sub // KROWS

    def body(t0, t1, t2, t3, idx_g, idx_s, zp, out, gi, si, rows, acc,
             sem_g, sem_s):
        core = lax.axis_index("c")
        sid = lax.axis_index("s")
        npr = PAT_PAD // 16
        row_base = sid * rows_per_sub
        for q_local, (ta, tb) in enumerate(((t0, t2), (t1, t3))):
            pltpu.sync_copy(zp.at[pl.ds(sid * npr, npr)],
                            acc.at[pl.ds(sid * npr, npr)])
            plsc.subcore_barrier()

            @pl.when(core == 0)
            def _():
                _edge_phase(idx_g, idx_s, row_base, n_chunks, gi, si, rows,
                            acc, ta, sem_g, sem_s)

            @pl.when(core == 1)
            def _():
                _edge_phase(idx_g, idx_s, row_base, n_chunks, gi, si, rows,
                            acc, tb, sem_g, sem_s)

            plsc.subcore_barrier()
            pltpu.sync_copy(acc.at[pl.ds(sid * npr, npr)],
                            out.at[2 * core + q_local, pl.ds(sid * npr, npr)])

    return pl.kernel(body, out_type=out_type, mesh=_mesh,
                     scratch_types=scratch, compiler_params=_SC_PARAMS)


PB = 2000
NPB = NP_N // PB


def _full(shape):
    return pl.BlockSpec(shape, lambda i: (0,) * len(shape))


def _rows(w):
    return pl.BlockSpec((PB, w), lambda i: (i, 0))


def _dot(a, b):
    return jnp.dot(a, b, preferred_element_type=f32)


def _node_dense_l1(a0, a1, c0, c1, x_raw, w_pat, b_pat, w_own, b_own,
                   wl, wr, bias, s, t, n, k_raw):
    def body(a0r, a1r, c0r, c1r, xr, wpr, bpr, wor, bor, wlr, wrr, br, sr, tr,
             o_ref, cnt_ref):
        cnt = c0r[:, 0:1] + c1r[:, 0:1]
        cnt_ref[...] = cnt
        mean_raw = (a0r[...] + a1r[...]) / jnp.maximum(cnt, 1.0)
        xm = _dot(mean_raw, wpr[...]) + bpr[...]
        x_own = bor[...]
        for j in range(k_raw):
            x_own = x_own + xr[:, j:j + 1] * wor[j:j + 1, :]
        o = _dot(xm, wlr[...]) + _dot(x_own, wrr[...]) + br[...]
        o_ref[...] = jnp.maximum(o * sr[...] + tr[...], 0.0)

    res = pl.pallas_call(
        body, grid=(1,),
        in_specs=[_full((n, 16)), _full((n, 16)), _full((n, DC)),
                  _full((n, DC)), _full((n, k_raw)), _full((16, H)),
                  _full((1, H)), _full((k_raw, H)), _full((1, H)),
                  _full((H, H)), _full((H, H)), _full((1, H)),
                  _full((1, H)), _full((1, H))],
        out_specs=[_full((n, H)), _full((n, 1))],
        out_shape=[jax.ShapeDtypeStruct((n, H), f32),
                   jax.ShapeDtypeStruct((n, 1), f32)],
    )(a0, a1, c0, c1, x_raw, w_pat, b_pat, w_own, b_own, wl, wr, bias, s, t)
    return res


def _node_dense_l2(a0, a1, cnt, x, wl, wr, bias, s, t, n):
    def body(a0r, a1r, cr, xr, wlr, wrr, br, sr, tr, o_ref):
        mean = (a0r[...] + a1r[...]) / jnp.maximum(cr[...], 1.0)
        o = _dot(mean, wlr[...]) + _dot(xr[...], wrr[...]) + br[...]
        o_ref[...] = jnp.maximum(o * sr[...] + tr[...], 0.0)

    return pl.pallas_call(
        body, grid=(1,),
        in_specs=[_full((n, H)), _full((n, H)), _full((n, 1)), _full((n, H)),
                  _full((H, H)), _full((H, H)), _full((1, H)), _full((1, H)),
                  _full((1, H))],
        out_specs=_full((n, H)),
        out_shape=jax.ShapeDtypeStruct((n, H), f32),
    )(a0, a1, cnt, x, wl, wr, bias, s, t)


def _pat_dense_l1(b0, b1, c0, c1, x_raw, w_pat, b_pat, w_lab, b_lab,
                  w_dis, b_dis, wl_l, wl_d, wr, bias, s, t):
    def body(b0r, b1r, c0r, c1r, xr, wpr, bpr, wlbr, blbr, wdr, bdr,
             wllr, wldr, wrr, br, sr, tr, o_ref, cnt_ref):
        cl = b0r[:, 1:2] + b1r[:, 1:2]
        cd = c0r[:, 2:3] + c1r[:, 2:3]
        cnt_ref[...] = jnp.concatenate([cl, cd], axis=1)
        mean_u = (b0r[:, 0:1] + b1r[:, 0:1]) / jnp.maximum(cl, 1.0)
        xl_mean = mean_u * wlbr[...] + blbr[...]
        cdm = jnp.maximum(cd, 1.0)
        d0 = (c0r[:, 0:1] + c1r[:, 0:1]) / cdm
        d1 = (c0r[:, 1:2] + c1r[:, 1:2]) / cdm
        xd_mean = d0 * wdr[0:1, :] + d1 * wdr[1:2, :] + bdr[...]
        x_own = _dot(xr[...], wpr[...]) + bpr[...]
        o = (_dot(xl_mean, wllr[...]) + _dot(xd_mean, wldr[...])
             + _dot(x_own, wrr[...]) + br[...])
        o_ref[...] = jnp.maximum(o * sr[...] + tr[...], 0.0)

    return pl.pallas_call(
        body, grid=(NPB,),
        in_specs=[_rows(DC)] * 4 + [_rows(16)]
        + [_full((16, H)), _full((1, H)), _full((1, H)), _full((1, H)),
           _full((2, H)), _full((1, H)), _full((H, H)), _full((H, H)),
           _full((H, H)), _full((1, H)), _full((1, H)), _full((1, H))],
        out_specs=[_rows(H), _rows(2)],
        out_shape=[jax.ShapeDtypeStruct((NP_N, H), f32),
                   jax.ShapeDtypeStruct((NP_N, 2), f32)],
    )(b0, b1, c0, c1, x_raw, w_pat, b_pat, w_lab, b_lab, w_dis, b_dis,
      wl_l, wl_d, wr, bias, s, t)


def _pat_dense_l2(bq, cq, cnt, xp, wl_l, wl_d, wr, bias, s, t):
    wl_ls = [wl_l[16 * q:16 * (q + 1)] for q in range(4)]
    wl_ds = [wl_d[16 * q:16 * (q + 1)] for q in range(4)]

    def body(b0r, b1r, b2r, b3r, c0r, c1r, c2r, c3r, cr, xr,
             wll0, wll1, wll2, wll3, wld0, wld1, wld2, wld3,
             wrr, br, sr, tr, o_ref):
        cl = jnp.maximum(cr[:, 0:1], 1.0)
        cd = jnp.maximum(cr[:, 1:2], 1.0)
        o = _dot(xr[...], wrr[...]) + br[...]
        for qr, wq in zip((b0r, b1r, b2r, b3r), (wll0, wll1, wll2, wll3)):
            o = o + _dot(qr[...] / cl, wq[...])
        for qr, wq in zip((c0r, c1r, c2r, c3r), (wld0, wld1, wld2, wld3)):
            o = o + _dot(qr[...] / cd, wq[...])
        o_ref[...] = jnp.maximum(o * sr[...] + tr[...], 0.0)

    return pl.pallas_call(
        body, grid=(NPB,),
        in_specs=[_rows(DB)] * 8 + [_rows(2), _rows(H)]
        + [_full((16, H))] * 8
        + [_full((H, H)), _full((1, H)), _full((1, H)), _full((1, H))],
        out_specs=_rows(H),
        out_shape=jax.ShapeDtypeStruct((NP_N, H), f32),
    )(*bq, *cq, cnt, xp, *wl_ls, *wl_ds, wr, bias, s, t)


def _head(xp, w1, b1, w2, b2):
    def body(x_ref, w1_ref, b1_ref, w2_ref, b2_ref, o_ref):
        h = jnp.maximum(_dot(x_ref[...], w1_ref[...]) + b1_ref[...], 0.0)
        logits = _dot(h, w2_ref[...]) + b2_ref[...]
        m = jnp.max(logits, axis=1, keepdims=True)
        lse = jnp.log(jnp.sum(jnp.exp(logits - m), axis=1, keepdims=True)) + m
        o_ref[...] = logits - lse

    return pl.pallas_call(
        body, grid=(NPB,),
        in_specs=[_rows(H), _full((H, H // 2)), _full((1, H // 2)),
                  _full((H // 2, OUT)), _full((1, OUT))],
        out_specs=_rows(OUT),
        out_shape=jax.ShapeDtypeStruct((NP_N, OUT), f32),
    )(xp, w1, b1, w2, b2)


_agg_ld1 = _make_agg_ld(16, True)
_agg_ld2 = _make_agg_ld(H, False)
_agg_pl1 = _make_agg_pat_small(EPL_PAD)
_agg_pd1 = _make_agg_pat_small(EPD_PAD)
_agg_pl2 = _make_agg_pat(EPL_PAD)
_agg_pd2 = _make_agg_pat(EPD_PAD)

_BN_INV = 1.0 / math.sqrt(1.0 + 1e-5)


def _pad_idx(idx, n_pad, fill):
    return jnp.concatenate(
        [idx, jnp.full((n_pad - idx.shape[0],), fill, i32)]).reshape(-1, 128)


def kernel(x_patient, x_lab, x_disease, src_pl, dst_pl, src_pd, dst_pd,
           W_pat, b_pat, W_lab, b_lab, W_dis, b_dis,
           c1_pl_Wl, c1_pl_bl, c1_pl_Wr, c1_pd_Wl, c1_pd_bl, c1_pd_Wr,
           c1_lp_Wl, c1_lp_bl, c1_lp_Wr, c1_dp_Wl, c1_dp_bl, c1_dp_Wr,
           c2_pl_Wl, c2_pl_bl, c2_pl_Wr, c2_pd_Wl, c2_pd_bl, c2_pd_Wr,
           c2_lp_Wl, c2_lp_bl, c2_lp_Wr, c2_dp_Wl, c2_dp_bl, c2_dp_Wr,
           bn1_p_g, bn1_p_b, bn1_l_g, bn1_l_b, bn1_d_g, bn1_d_b,
           bn2_p_g, bn2_p_b, bn2_l_g, bn2_l_b, bn2_d_g, bn2_d_b,
           W_fc1, b_fc1, W_fc2, b_fc2):
    row = lambda v: v.reshape(1, -1)

    spl_gA = _pad_idx(src_pl, EPL_PAD, 0)
    dpl_sA = _pad_idx(dst_pl, EPL_PAD, NL_N)
    dpl_gB = _pad_idx(dst_pl, EPL_PAD, 0)
    spl_sB = _pad_idx(src_pl, EPL_PAD, NP_N)
    spd_gA = _pad_idx(src_pd, EPD_PAD, 0)
    dpd_sA = _pad_idx(dst_pd, EPD_PAD, ND_N)
    dpd_gC = _pad_idx(dst_pd, EPD_PAD, 0)
    spd_sC = _pad_idx(src_pd, EPD_PAD, NP_N)

    z16l = jnp.zeros((LAB_PAD, 16), f32)
    z16d = jnp.zeros((DIS_PAD, 16), f32)
    z64l = jnp.zeros((LAB_PAD, H), f32)
    z64d = jnp.zeros((DIS_PAD, H), f32)
    zc8l = jnp.zeros((LAB_PAD, DC), f32)
    zc8d = jnp.zeros((DIS_PAD, DC), f32)
    zp16 = jnp.zeros((PAT_PAD, DB), f32)
    zp8 = jnp.zeros((PAT_PAD, DC), f32)
    ones_rows = jnp.ones((CHUNK, DC), f32)

    tl8 = jnp.concatenate(
        [x_lab, jnp.ones((NL_N, 1), f32), jnp.zeros((NL_N, DC - 2), f32)],
        axis=1)
    td8 = jnp.concatenate(
        [x_disease, jnp.ones((ND_N, 1), f32), jnp.zeros((ND_N, DC - 3), f32)],
        axis=1)

    out_l, out_d, oc_l, oc_d = _agg_ld1(
        x_patient, spl_gA, dpl_sA, spd_gA, dpd_sA, z16l, z16d, zc8l, zc8d,
        ones_rows)
    out_b = _agg_pl1(tl8, dpl_gB, spl_sB, zp8)
    out_c = _agg_pd1(td8, dpd_gC, spd_sC, zp8)

    s_l1, s_d1, s_p1 = (row(bn1_l_g) * _BN_INV, row(bn1_d_g) * _BN_INV,
                        row(bn1_p_g) * _BN_INV)
    x_l, cnt_l = _node_dense_l1(
        out_l[0, :NL_N], out_l[1, :NL_N], oc_l[0, :NL_N], oc_l[1, :NL_N],
        x_lab, W_pat, row(b_pat), W_lab, row(b_lab), c1_pl_Wl, c1_pl_Wr,
        row(c1_pl_bl), s_l1, row(bn1_l_b), NL_N, 1)
    x_d, cnt_d = _node_dense_l1(
        out_d[0, :ND_N], out_d[1, :ND_N], oc_d[0, :ND_N], oc_d[1, :ND_N],
        x_disease, W_pat, row(b_pat), W_dis, row(b_dis), c1_pd_Wl, c1_pd_Wr,
        row(c1_pd_bl), s_d1, row(bn1_d_b), ND_N, 2)
    x_p, cnt_p = _pat_dense_l1(
        out_b[0, :NP_N], out_b[1, :NP_N], out_c[0, :NP_N], out_c[1, :NP_N],
        x_patient, W_pat, row(b_pat), row(W_lab[0]), row(b_lab), W_dis,
        row(b_dis), c1_lp_Wl, c1_dp_Wl, c1_lp_Wr + c1_dp_Wr,
        row(c1_lp_bl + c1_dp_bl), s_p1, row(bn1_p_b))

    tls = [x_l[:, 16 * q:16 * (q + 1)] for q in range(4)]
    tds = [x_d[:, 16 * q:16 * (q + 1)] for q in range(4)]

    out_l2, out_d2 = _agg_ld2(x_p, spl_gA, dpl_sA, spd_gA, dpd_sA, z64l, z64d)
    out_b2 = _agg_pl2(*tls, dpl_gB, spl_sB, zp16)
    out_c2 = _agg_pd2(*tds, dpd_gC, spd_sC, zp16)

    s_l2, s_d2, s_p2 = (row(bn2_l_g) * _BN_INV, row(bn2_d_g) * _BN_INV,
                        row(bn2_p_g) * _BN_INV)
    x_l = _node_dense_l2(out_l2[0, :NL_N], out_l2[1, :NL_N], cnt_l, x_l,
                         c2_pl_Wl, c2_pl_Wr, row(c2_pl_bl), s_l2,
                         row(bn2_l_b), NL_N)
    x_d = _node_dense_l2(out_d2[0, :ND_N], out_d2[1, :ND_N], cnt_d, x_d,
                         c2_pd_Wl, c2_pd_Wr, row(c2_pd_bl), s_d2,
                         row(bn2_d_b), ND_N)
    bq = [out_b2[q, :NP_N] for q in range(4)]
    cq = [out_c2[q, :NP_N] for q in range(4)]
    x_p = _pat_dense_l2(bq, cq, cnt_p, x_p, c2_lp_Wl, c2_dp_Wl,
                        c2_lp_Wr + c2_dp_Wr, row(c2_lp_bl + c2_dp_bl),
                        s_p2, row(bn2_p_b))

    return _head(x_p, W_fc1, row(b_fc1), W_fc2, row(b_fc2))


# speedup vs baseline: 5.5924x; 1.1115x over previous
import functools
import math

import jax
import jax.numpy as jnp
from jax import lax
from jax.experimental import pallas as pl
from jax.experimental.pallas import tpu as pltpu
from jax.experimental.pallas import tpu_sc as plsc

NP_N, NL_N, ND_N = 50000, 5000, 500
EPL_PAD, EPD_PAD = 819200, 229376
H = 64
OUT = 50
LAB_PAD, DIS_PAD, PAT_PAD = 5120, 512, 50048
DB = 16
DC = 8
CHUNK = 1024
KROWS = CHUNK // 128

_mesh = plsc.VectorSubcoreMesh(core_axis_name="c", subcore_axis_name="s")
_SC_PARAMS = pltpu.CompilerParams(use_tc_tiling_on_sc=False)

f32 = jnp.float32
i32 = jnp.int32


def _edge_phase(idx_g_hbm, idx_s_hbm, row_base, n_chunks, gi, si, rows, acc,
                table_ref, sem_g, sem_s, dummy, chunk):
    krows = chunk // 128

    def load_idx(t, half):
        rb = row_base + t * krows
        pltpu.sync_copy(idx_g_hbm.at[pl.ds(rb, krows)],
                        gi.at[pl.ds(half * krows, krows)])
        pltpu.sync_copy(idx_s_hbm.at[pl.ds(rb, krows)],
                        si.at[pl.ds(half * krows, krows)])

    def fire_gathers(half):
        for j in range(krows):
            pltpu.async_copy(
                table_ref.at[gi.at[half * krows + j]],
                rows.at[pl.ds(half * chunk + j * 128, 128)], sem_g)

    def fire_scatters(half):
        for j in range(krows):
            pltpu.async_copy(
                rows.at[pl.ds(half * chunk + j * 128, 128)],
                acc.at[si.at[half * krows + j]], sem_s, add=True)

    def drain(sem):
        pltpu.make_async_copy(dummy.at[pl.ds(0, chunk)],
                              rows.at[pl.ds(0, chunk)], sem).wait()

    load_idx(0, 0)
    fire_gathers(0)

    def body(t, carry):
        for p in (0, 1):
            @pl.when(lax.rem(t, 2) == p)
            def _():
                o = 1 - p
                load_idx(t + 1, o)

                @pl.when(t >= 1)
                def _():
                    drain(sem_s)

                fire_gathers(o)
                drain(sem_g)
                fire_scatters(p)
        return carry

    lax.fori_loop(0, n_chunks - 1, body, 0)
    drain(sem_g)
    fire_scatters((n_chunks - 1) % 2)
    for _ in range(min(2, n_chunks)):
        drain(sem_s)


def _count_phase(idx_s_hbm, row_base, n_chunks, si, ones_rows, acc, sem_s):
    def body(t, carry):
        rb = row_base + t * KROWS
        pltpu.sync_copy(idx_s_hbm.at[pl.ds(rb, KROWS)],
                        si.at[pl.ds(0, KROWS)])
        puts = []
        for j in range(KROWS):
            puts.append(pltpu.async_copy(
                ones_rows.at[pl.ds(j * 128, 128)], acc.at[si.at[j]], sem_s,
                add=True))
        for d in puts:
            d.wait()
        return carry

    lax.fori_loop(0, n_chunks, body, 0)


def _make_agg_ld(width, with_counts, chunk):
    krows = chunk // 128
    out_type = [
        jax.ShapeDtypeStruct((2, LAB_PAD, width), f32),
        jax.ShapeDtypeStruct((2, DIS_PAD, width), f32),
    ]
    scratch = [
        pltpu.VMEM((2 * krows, 128), i32),
        pltpu.VMEM((2 * krows, 128), i32),
        pltpu.VMEM((2 * chunk, width), f32),
        pltpu.VMEM_SHARED((LAB_PAD, width), f32),
        pltpu.VMEM_SHARED((DIS_PAD, width), f32),
        pltpu.SemaphoreType.DMA,
        pltpu.SemaphoreType.DMA,
    ]
    if with_counts:
        out_type += [
            jax.ShapeDtypeStruct((2, LAB_PAD, DC), f32),
            jax.ShapeDtypeStruct((2, DIS_PAD, DC), f32),
        ]
        scratch += [
            pltpu.VMEM((CHUNK, DC), f32),
            pltpu.VMEM_SHARED((LAB_PAD, DC), f32),
            pltpu.VMEM_SHARED((DIS_PAD, DC), f32),
        ]

    rows_per_tile_pl = EPL_PAD // 32 // 128
    rows_per_tile_pd = EPD_PAD // 32 // 128

    def body(*refs):
        if with_counts:
            (tp, spl_g, dpl_s, spd_g, dpd_s, zl, zd, zcl, zcd, ones_hbm,
             out_l, out_d, out_cl, out_cd,
             gi, si, rows, acc_l, acc_d, sem_g, sem_s,
             ones_rows, acc_cl, acc_cd) = refs
        else:
            (tp, spl_g, dpl_s, spd_g, dpd_s, zl, zd, out_l, out_d,
             gi, si, rows, acc_l, acc_d, sem_g, sem_s) = refs
        core = lax.axis_index("c")
        sid = lax.axis_index("s")
        tile = core * 16 + sid
        nl = LAB_PAD // 16
        nd = DIS_PAD // 16
        pltpu.sync_copy(zl.at[pl.ds(sid * nl, nl)],
                        acc_l.at[pl.ds(sid * nl, nl)])
        pltpu.sync_copy(zd.at[pl.ds(sid * nd, nd)],
                        acc_d.at[pl.ds(sid * nd, nd)])
        if with_counts:
            pltpu.sync_copy(zcl.at[pl.ds(sid * nl, nl)],
                            acc_cl.at[pl.ds(sid * nl, nl)])
            pltpu.sync_copy(zcd.at[pl.ds(sid * nd, nd)],
                            acc_cd.at[pl.ds(sid * nd, nd)])
            pltpu.sync_copy(ones_hbm, ones_rows)
        plsc.subcore_barrier()
        _edge_phase(spl_g, dpl_s, tile * rows_per_tile_pl,
                    rows_per_tile_pl // krows, gi, si, rows, acc_l, tp,
                    sem_g, sem_s, zl, chunk)
        _edge_phase(spd_g, dpd_s, tile * rows_per_tile_pd,
                    rows_per_tile_pd // krows, gi, si, rows, acc_d, tp,
                    sem_g, sem_s, zl, chunk)
        if with_counts:
            _count_phase(dpl_s, tile * rows_per_tile_pl,
                         rows_per_tile_pl // KROWS, si, ones_rows, acc_cl,
                         sem_s)
            _count_phase(dpd_s, tile * rows_per_tile_pd,
                         rows_per_tile_pd // KROWS, si, ones_rows, acc_cd,
                         sem_s)
        plsc.subcore_barrier()
        pltpu.sync_copy(acc_l.at[pl.ds(sid * nl, nl)],
                        out_l.at[core, pl.ds(sid * nl, nl)])
        pltpu.sync_copy(acc_d.at[pl.ds(sid * nd, nd)],
                        out_d.at[core, pl.ds(sid * nd, nd)])
        if with_counts:
            pltpu.sync_copy(acc_cl.at[pl.ds(sid * nl, nl)],
                            out_cl.at[core, pl.ds(sid * nl, nl)])
            pltpu.sync_copy(acc_cd.at[pl.ds(sid * nd, nd)],
                            out_cd.at[core, pl.ds(sid * nd, nd)])

    return pl.kernel(body, out_type=out_type, mesh=_mesh,
                     scratch_types=scratch, compiler_params=_SC_PARAMS)


def _make_agg_pat_small(n_edges_pad):
    out_type = jax.ShapeDtypeStruct((2, PAT_PAD, DC), f32)
    scratch = [
        pltpu.VMEM((2 * KROWS, 128), i32),
        pltpu.VMEM((2 * KROWS, 128), i32),
        pltpu.VMEM((2 * CHUNK, DC), f32),
        pltpu.VMEM_SHARED((PAT_PAD, DC), f32),
        pltpu.SemaphoreType.DMA,
        pltpu.SemaphoreType.DMA,
    ]
    rows_per_tile = n_edges_pad // 32 // 128

    def body(t, idx_g, idx_s, zp, out, gi, si, rows, acc, sem_g, sem_s):
        core = lax.axis_index("c")
        sid = lax.axis_index("s")
        tile = core * 16 + sid
        npr = PAT_PAD // 16
        pltpu.sync_copy(zp.at[pl.ds(sid * npr, npr)],
                        acc.at[pl.ds(sid * npr, npr)])
        plsc.subcore_barrier()
        _edge_phase(idx_g, idx_s, tile * rows_per_tile,
                    rows_per_tile // KROWS, gi, si, rows, acc, t,
                    sem_g, sem_s, zp, CHUNK)
        plsc.subcore_barrier()
        pltpu.sync_copy(acc.at[pl.ds(sid * npr, npr)],
                        out.at[core, pl.ds(sid * npr, npr)])

    return pl.kernel(body, out_type=out_type, mesh=_mesh,
                     scratch_types=scratch, compiler_params=_SC_PARAMS)


def _make_agg_pat(n_edges_pad):
    out_type = jax.ShapeDtypeStruct((4, PAT_PAD, DB), f32)
    scratch = [
        pltpu.VMEM((2 * KROWS, 128), i32),
        pltpu.VMEM((2 * KROWS, 128), i32),
        pltpu.VMEM((2 * CHUNK, DB), f32),
        pltpu.VMEM_SHARED((PAT_PAD, DB), f32),
        pltpu.SemaphoreType.DMA,
        pltpu.SemaphoreType.DMA,
    ]
    rows_per_sub = n_edges_pad // 16 // 128
    n_chunks = rows_per_sub // KROWS

    def body(t0, t1, t2, t3, idx_g, idx_s, zp, out, gi, si, rows, acc,
             sem_g, sem_s):
        core = lax.axis_index("c")
        sid = lax.axis_index("s")
        npr = PAT_PAD // 16
        row_base = sid * rows_per_sub
        for q_local, (ta, tb) in enumerate(((t0, t2), (t1, t3))):
            pltpu.sync_copy(zp.at[pl.ds(sid * npr, npr)],
                            acc.at[pl.ds(sid * npr, npr)])
            plsc.subcore_barrier()

            @pl.when(core == 0)
            def _():
                _edge_phase(idx_g, idx_s, row_base, n_chunks, gi, si, rows,
                            acc, ta, sem_g, sem_s, zp, CHUNK)

            @pl.when(core == 1)
            def _():
                _edge_phase(idx_g, idx_s, row_base, n_chunks, gi, si, rows,
                            acc, tb, sem_g, sem_s, zp, CHUNK)

            plsc.subcore_barrier()
            pltpu.sync_copy(acc.at[pl.ds(sid * npr, npr)],
                            out.at[2 * core + q_local, pl.ds(sid * npr, npr)])

    return pl.kernel(body, out_type=out_type, mesh=_mesh,
                     scratch_types=scratch, compiler_params=_SC_PARAMS)


PB = 2000
NPB = NP_N // PB


def _full(shape):
    return pl.BlockSpec(shape, lambda i: (0,) * len(shape))


def _rows(w):
    return pl.BlockSpec((PB, w), lambda i: (i, 0))


def _dot(a, b):
    return jnp.dot(a, b, preferred_element_type=f32)


def _node_dense_l1(a0, a1, c0, c1, x_raw, w_pat, b_pat, w_own, b_own,
                   wl, wr, bias, s, t, n, k_raw):
    def body(a0r, a1r, c0r, c1r, xr, wpr, bpr, wor, bor, wlr, wrr, br, sr, tr,
             o_ref, cnt_ref):
        cnt = c0r[:, 0:1] + c1r[:, 0:1]
        cnt_ref[...] = cnt
        mean_raw = (a0r[...] + a1r[...]) / jnp.maximum(cnt, 1.0)
        xm = _dot(mean_raw, wpr[...]) + bpr[...]
        x_own = bor[...]
        for j in range(k_raw):
            x_own = x_own + xr[:, j:j + 1] * wor[j:j + 1, :]
        o = _dot(xm, wlr[...]) + _dot(x_own, wrr[...]) + br[...]
        o_ref[...] = jnp.maximum(o * sr[...] + tr[...], 0.0)

    res = pl.pallas_call(
        body, grid=(1,),
        in_specs=[_full((n, 16)), _full((n, 16)), _full((n, DC)),
                  _full((n, DC)), _full((n, k_raw)), _full((16, H)),
                  _full((1, H)), _full((k_raw, H)), _full((1, H)),
                  _full((H, H)), _full((H, H)), _full((1, H)),
                  _full((1, H)), _full((1, H))],
        out_specs=[_full((n, H)), _full((n, 1))],
        out_shape=[jax.ShapeDtypeStruct((n, H), f32),
                   jax.ShapeDtypeStruct((n, 1), f32)],
    )(a0, a1, c0, c1, x_raw, w_pat, b_pat, w_own, b_own, wl, wr, bias, s, t)
    return res


def _node_dense_l2(a0, a1, cnt, x, wl, wr, bias, s, t, n):
    def body(a0r, a1r, cr, xr, wlr, wrr, br, sr, tr, o_ref):
        mean = (a0r[...] + a1r[...]) / jnp.maximum(cr[...], 1.0)
        o = _dot(mean, wlr[...]) + _dot(xr[...], wrr[...]) + br[...]
        o_ref[...] = jnp.maximum(o * sr[...] + tr[...], 0.0)

    return pl.pallas_call(
        body, grid=(1,),
        in_specs=[_full((n, H)), _full((n, H)), _full((n, 1)), _full((n, H)),
                  _full((H, H)), _full((H, H)), _full((1, H)), _full((1, H)),
                  _full((1, H))],
        out_specs=_full((n, H)),
        out_shape=jax.ShapeDtypeStruct((n, H), f32),
    )(a0, a1, cnt, x, wl, wr, bias, s, t)


def _pat_dense_l1(b0, b1, c0, c1, x_raw, w_pat, b_pat, w_lab, b_lab,
                  w_dis, b_dis, wl_l, wl_d, wr, bias, s, t):
    def body(b0r, b1r, c0r, c1r, xr, wpr, bpr, wlbr, blbr, wdr, bdr,
             wllr, wldr, wrr, br, sr, tr, o_ref, cnt_ref):
        cl = b0r[:, 1:2] + b1r[:, 1:2]
        cd = c0r[:, 2:3] + c1r[:, 2:3]
        cnt_ref[...] = jnp.concatenate([cl, cd], axis=1)
        mean_u = (b0r[:, 0:1] + b1r[:, 0:1]) / jnp.maximum(cl, 1.0)
        xl_mean = mean_u * wlbr[...] + blbr[...]
        cdm = jnp.maximum(cd, 1.0)
        d0 = (c0r[:, 0:1] + c1r[:, 0:1]) / cdm
        d1 = (c0r[:, 1:2] + c1r[:, 1:2]) / cdm
        xd_mean = d0 * wdr[0:1, :] + d1 * wdr[1:2, :] + bdr[...]
        x_own = _dot(xr[...], wpr[...]) + bpr[...]
        o = (_dot(xl_mean, wllr[...]) + _dot(xd_mean, wldr[...])
             + _dot(x_own, wrr[...]) + br[...])
        o_ref[...] = jnp.maximum(o * sr[...] + tr[...], 0.0)

    return pl.pallas_call(
        body, grid=(NPB,),
        in_specs=[_rows(DC)] * 4 + [_rows(16)]
        + [_full((16, H)), _full((1, H)), _full((1, H)), _full((1, H)),
           _full((2, H)), _full((1, H)), _full((H, H)), _full((H, H)),
           _full((H, H)), _full((1, H)), _full((1, H)), _full((1, H))],
        out_specs=[_rows(H), _rows(2)],
        out_shape=[jax.ShapeDtypeStruct((NP_N, H), f32),
                   jax.ShapeDtypeStruct((NP_N, 2), f32)],
    )(b0, b1, c0, c1, x_raw, w_pat, b_pat, w_lab, b_lab, w_dis, b_dis,
      wl_l, wl_d, wr, bias, s, t)


def _pat_dense_l2(bq, cq, cnt, xp, wl_l, wl_d, wr, bias, s, t):
    wl_ls = [wl_l[16 * q:16 * (q + 1)] for q in range(4)]
    wl_ds = [wl_d[16 * q:16 * (q + 1)] for q in range(4)]

    def body(b0r, b1r, b2r, b3r, c0r, c1r, c2r, c3r, cr, xr,
             wll0, wll1, wll2, wll3, wld0, wld1, wld2, wld3,
             wrr, br, sr, tr, o_ref):
        cl = jnp.maximum(cr[:, 0:1], 1.0)
        cd = jnp.maximum(cr[:, 1:2], 1.0)
        o = _dot(xr[...], wrr[...]) + br[...]
        for qr, wq in zip((b0r, b1r, b2r, b3r), (wll0, wll1, wll2, wll3)):
            o = o + _dot(qr[...] / cl, wq[...])
        for qr, wq in zip((c0r, c1r, c2r, c3r), (wld0, wld1, wld2, wld3)):
            o = o + _dot(qr[...] / cd, wq[...])
        o_ref[...] = jnp.maximum(o * sr[...] + tr[...], 0.0)

    return pl.pallas_call(
        body, grid=(NPB,),
        in_specs=[_rows(DB)] * 8 + [_rows(2), _rows(H)]
        + [_full((16, H))] * 8
        + [_full((H, H)), _full((1, H)), _full((1, H)), _full((1, H))],
        out_specs=_rows(H),
        out_shape=jax.ShapeDtypeStruct((NP_N, H), f32),
    )(*bq, *cq, cnt, xp, *wl_ls, *wl_ds, wr, bias, s, t)


def _head(xp, w1, b1, w2, b2):
    def body(x_ref, w1_ref, b1_ref, w2_ref, b2_ref, o_ref):
        h = jnp.maximum(_dot(x_ref[...], w1_ref[...]) + b1_ref[...], 0.0)
        logits = _dot(h, w2_ref[...]) + b2_ref[...]
        m = jnp.max(logits, axis=1, keepdims=True)
        lse = jnp.log(jnp.sum(jnp.exp(logits - m), axis=1, keepdims=True)) + m
        o_ref[...] = logits - lse

    return pl.pallas_call(
        body, grid=(NPB,),
        in_specs=[_rows(H), _full((H, H // 2)), _full((1, H // 2)),
                  _full((H // 2, OUT)), _full((1, OUT))],
        out_specs=_rows(OUT),
        out_shape=jax.ShapeDtypeStruct((NP_N, OUT), f32),
    )(xp, w1, b1, w2, b2)


_agg_ld1 = _make_agg_ld(16, True, 1024)
_agg_ld2 = _make_agg_ld(H, False, 512)
_agg_pl1 = _make_agg_pat_small(EPL_PAD)
_agg_pd1 = _make_agg_pat_small(EPD_PAD)
_agg_pl2 = _make_agg_pat(EPL_PAD)
_agg_pd2 = _make_agg_pat(EPD_PAD)

_BN_INV = 1.0 / math.sqrt(1.0 + 1e-5)


def _pad_idx(idx, n_pad, fill):
    return jnp.concatenate(
        [idx, jnp.full((n_pad - idx.shape[0],), fill, i32)]).reshape(-1, 128)


def kernel(x_patient, x_lab, x_disease, src_pl, dst_pl, src_pd, dst_pd,
           W_pat, b_pat, W_lab, b_lab, W_dis, b_dis,
           c1_pl_Wl, c1_pl_bl, c1_pl_Wr, c1_pd_Wl, c1_pd_bl, c1_pd_Wr,
           c1_lp_Wl, c1_lp_bl, c1_lp_Wr, c1_dp_Wl, c1_dp_bl, c1_dp_Wr,
           c2_pl_Wl, c2_pl_bl, c2_pl_Wr, c2_pd_Wl, c2_pd_bl, c2_pd_Wr,
           c2_lp_Wl, c2_lp_bl, c2_lp_Wr, c2_dp_Wl, c2_dp_bl, c2_dp_Wr,
           bn1_p_g, bn1_p_b, bn1_l_g, bn1_l_b, bn1_d_g, bn1_d_b,
           bn2_p_g, bn2_p_b, bn2_l_g, bn2_l_b, bn2_d_g, bn2_d_b,
           W_fc1, b_fc1, W_fc2, b_fc2):
    row = lambda v: v.reshape(1, -1)

    spl_gA = _pad_idx(src_pl, EPL_PAD, 0)
    dpl_sA = _pad_idx(dst_pl, EPL_PAD, NL_N)
    dpl_gB = _pad_idx(dst_pl, EPL_PAD, 0)
    spl_sB = _pad_idx(src_pl, EPL_PAD, NP_N)
    spd_gA = _pad_idx(src_pd, EPD_PAD, 0)
    dpd_sA = _pad_idx(dst_pd, EPD_PAD, ND_N)
    dpd_gC = _pad_idx(dst_pd, EPD_PAD, 0)
    spd_sC = _pad_idx(src_pd, EPD_PAD, NP_N)

    z16l = jnp.zeros((LAB_PAD, 16), f32)
    z16d = jnp.zeros((DIS_PAD, 16), f32)
    z64l = jnp.zeros((LAB_PAD, H), f32)
    z64d = jnp.zeros((DIS_PAD, H), f32)
    zc8l = jnp.zeros((LAB_PAD, DC), f32)
    zc8d = jnp.zeros((DIS_PAD, DC), f32)
    zp16 = jnp.zeros((PAT_PAD, DB), f32)
    zp8 = jnp.zeros((PAT_PAD, DC), f32)
    ones_rows = jnp.ones((CHUNK, DC), f32)

    tl8 = jnp.concatenate(
        [x_lab, jnp.ones((NL_N, 1), f32), jnp.zeros((NL_N, DC - 2), f32)],
        axis=1)
    td8 = jnp.concatenate(
        [x_disease, jnp.ones((ND_N, 1), f32), jnp.zeros((ND_N, DC - 3), f32)],
        axis=1)

    out_l, out_d, oc_l, oc_d = _agg_ld1(
        x_patient, spl_gA, dpl_sA, spd_gA, dpd_sA, z16l, z16d, zc8l, zc8d,
        ones_rows)
    out_b = _agg_pl1(tl8, dpl_gB, spl_sB, zp8)
    out_c = _agg_pd1(td8, dpd_gC, spd_sC, zp8)

    s_l1, s_d1, s_p1 = (row(bn1_l_g) * _BN_INV, row(bn1_d_g) * _BN_INV,
                        row(bn1_p_g) * _BN_INV)
    x_l, cnt_l = _node_dense_l1(
        out_l[0, :NL_N], out_l[1, :NL_N], oc_l[0, :NL_N], oc_l[1, :NL_N],
        x_lab, W_pat, row(b_pat), W_lab, row(b_lab), c1_pl_Wl, c1_pl_Wr,
        row(c1_pl_bl), s_l1, row(bn1_l_b), NL_N, 1)
    x_d, cnt_d = _node_dense_l1(
        out_d[0, :ND_N], out_d[1, :ND_N], oc_d[0, :ND_N], oc_d[1, :ND_N],
        x_disease, W_pat, row(b_pat), W_dis, row(b_dis), c1_pd_Wl, c1_pd_Wr,
        row(c1_pd_bl), s_d1, row(bn1_d_b), ND_N, 2)
    x_p, cnt_p = _pat_dense_l1(
        out_b[0, :NP_N], out_b[1, :NP_N], out_c[0, :NP_N], out_c[1, :NP_N],
        x_patient, W_pat, row(b_pat), row(W_lab[0]), row(b_lab), W_dis,
        row(b_dis), c1_lp_Wl, c1_dp_Wl, c1_lp_Wr + c1_dp_Wr,
        row(c1_lp_bl + c1_dp_bl), s_p1, row(bn1_p_b))

    tls = [x_l[:, 16 * q:16 * (q + 1)] for q in range(4)]
    tds = [x_d[:, 16 * q:16 * (q + 1)] for q in range(4)]

    out_l2, out_d2 = _agg_ld2(x_p, spl_gA, dpl_sA, spd_gA, dpd_sA, z64l, z64d)
    out_b2 = _agg_pl2(*tls, dpl_gB, spl_sB, zp16)
    out_c2 = _agg_pd2(*tds, dpd_gC, spd_sC, zp16)

    s_l2, s_d2, s_p2 = (row(bn2_l_g) * _BN_INV, row(bn2_d_g) * _BN_INV,
                        row(bn2_p_g) * _BN_INV)
    x_l = _node_dense_l2(out_l2[0, :NL_N], out_l2[1, :NL_N], cnt_l, x_l,
                         c2_pl_Wl, c2_pl_Wr, row(c2_pl_bl), s_l2,
                         row(bn2_l_b), NL_N)
    x_d = _node_dense_l2(out_d2[0, :ND_N], out_d2[1, :ND_N], cnt_d, x_d,
                         c2_pd_Wl, c2_pd_Wr, row(c2_pd_bl), s_d2,
                         row(bn2_d_b), ND_N)
    bq = [out_b2[q, :NP_N] for q in range(4)]
    cq = [out_c2[q, :NP_N] for q in range(4)]
    x_p = _pat_dense_l2(bq, cq, cnt_p, x_p, c2_lp_Wl, c2_dp_Wl,
                        c2_lp_Wr + c2_dp_Wr, row(c2_lp_bl + c2_dp_bl),
                        s_p2, row(bn2_p_b))

    return _head(x_p, W_fc1, row(b_fc1), W_fc2, row(b_fc2))


# speedup vs baseline: 5.6015x; 1.0016x over previous
import functools
import math

import jax
import jax.numpy as jnp
from jax import lax
from jax.experimental import pallas as pl
from jax.experimental.pallas import tpu as pltpu
from jax.experimental.pallas import tpu_sc as plsc

NP_N, NL_N, ND_N = 50000, 5000, 500
EPL_PAD, EPD_PAD = 819200, 229376
H = 64
OUT = 50
LAB_PAD, DIS_PAD, PAT_PAD = 5120, 512, 50048
DB = 16
DC = 8
CHUNK = 1024
KROWS = CHUNK // 128

_mesh = plsc.VectorSubcoreMesh(core_axis_name="c", subcore_axis_name="s")
_SC_PARAMS = pltpu.CompilerParams(use_tc_tiling_on_sc=False)

f32 = jnp.float32
i32 = jnp.int32


def _edge_phase(idx_g_hbm, idx_s_hbm, row_base, n_chunks, gi, si, rows, acc,
                table_ref, sem_g, sem_s, dummy, chunk):
    def load_idx(t, half):
        eb = row_base + t * chunk
        pltpu.sync_copy(idx_g_hbm.at[pl.ds(eb, chunk)], gi.at[half])
        pltpu.sync_copy(idx_s_hbm.at[pl.ds(eb, chunk)], si.at[half])

    def fire_gathers(half):
        pltpu.async_copy(table_ref.at[gi.at[half]],
                         rows.at[pl.ds(half * chunk, chunk)], sem_g)

    def fire_scatters(half):
        pltpu.async_copy(rows.at[pl.ds(half * chunk, chunk)],
                         acc.at[si.at[half]], sem_s, add=True)

    def drain(sem):
        pltpu.make_async_copy(dummy.at[pl.ds(0, chunk)],
                              rows.at[pl.ds(0, chunk)], sem).wait()

    load_idx(0, 0)
    fire_gathers(0)

    def body(t, carry):
        for p in (0, 1):
            @pl.when(lax.rem(t, 2) == p)
            def _():
                o = 1 - p
                load_idx(t + 1, o)

                @pl.when(t >= 1)
                def _():
                    drain(sem_s)

                fire_gathers(o)
                drain(sem_g)
                fire_scatters(p)
        return carry

    lax.fori_loop(0, n_chunks - 1, body, 0)
    drain(sem_g)
    fire_scatters((n_chunks - 1) % 2)
    for _ in range(min(2, n_chunks)):
        drain(sem_s)


def _count_phase(idx_s_hbm, edge_base, n_chunks, si, ones_rows, acc, sem_s):
    def body(t, carry):
        eb = edge_base + t * CHUNK
        pltpu.sync_copy(idx_s_hbm.at[pl.ds(eb, CHUNK)], si.at[0])
        pltpu.async_copy(ones_rows, acc.at[si.at[0]], sem_s,
                         add=True).wait()
        return carry

    lax.fori_loop(0, n_chunks, body, 0)


def _make_agg_ld(width, with_counts, chunk):
    out_type = [
        jax.ShapeDtypeStruct((2, LAB_PAD, width), f32),
        jax.ShapeDtypeStruct((2, DIS_PAD, width), f32),
    ]
    scratch = [
        pltpu.VMEM((2, chunk), i32),
        pltpu.VMEM((2, chunk), i32),
        pltpu.VMEM((2 * chunk, width), f32),
        pltpu.VMEM_SHARED((LAB_PAD, width), f32),
        pltpu.VMEM_SHARED((DIS_PAD, width), f32),
        pltpu.SemaphoreType.DMA,
        pltpu.SemaphoreType.DMA,
    ]
    if with_counts:
        out_type += [
            jax.ShapeDtypeStruct((2, LAB_PAD, DC), f32),
            jax.ShapeDtypeStruct((2, DIS_PAD, DC), f32),
        ]
        scratch += [
            pltpu.VMEM((CHUNK, DC), f32),
            pltpu.VMEM_SHARED((LAB_PAD, DC), f32),
            pltpu.VMEM_SHARED((DIS_PAD, DC), f32),
        ]

    ept_pl = EPL_PAD // 32
    ept_pd = EPD_PAD // 32

    def body(*refs):
        if with_counts:
            (tp, spl_g, dpl_s, spd_g, dpd_s, zl, zd, zcl, zcd, ones_hbm,
             out_l, out_d, out_cl, out_cd,
             gi, si, rows, acc_l, acc_d, sem_g, sem_s,
             ones_rows, acc_cl, acc_cd) = refs
        else:
            (tp, spl_g, dpl_s, spd_g, dpd_s, zl, zd, out_l, out_d,
             gi, si, rows, acc_l, acc_d, sem_g, sem_s) = refs
        core = lax.axis_index("c")
        sid = lax.axis_index("s")
        tile = core * 16 + sid
        nl = LAB_PAD // 16
        nd = DIS_PAD // 16
        pltpu.sync_copy(zl.at[pl.ds(sid * nl, nl)],
                        acc_l.at[pl.ds(sid * nl, nl)])
        pltpu.sync_copy(zd.at[pl.ds(sid * nd, nd)],
                        acc_d.at[pl.ds(sid * nd, nd)])
        if with_counts:
            pltpu.sync_copy(zcl.at[pl.ds(sid * nl, nl)],
                            acc_cl.at[pl.ds(sid * nl, nl)])
            pltpu.sync_copy(zcd.at[pl.ds(sid * nd, nd)],
                            acc_cd.at[pl.ds(sid * nd, nd)])
            pltpu.sync_copy(ones_hbm, ones_rows)
        plsc.subcore_barrier()
        _edge_phase(spl_g, dpl_s, tile * ept_pl, ept_pl // chunk,
                    gi, si, rows, acc_l, tp, sem_g, sem_s, zl, chunk)
        _edge_phase(spd_g, dpd_s, tile * ept_pd, ept_pd // chunk,
                    gi, si, rows, acc_d, tp, sem_g, sem_s, zl, chunk)
        if with_counts:
            _count_phase(dpl_s, tile * ept_pl, ept_pl // CHUNK,
                         si, ones_rows, acc_cl, sem_s)
            _count_phase(dpd_s, tile * ept_pd, ept_pd // CHUNK,
                         si, ones_rows, acc_cd, sem_s)
        plsc.subcore_barrier()
        pltpu.sync_copy(acc_l.at[pl.ds(sid * nl, nl)],
                        out_l.at[core, pl.ds(sid * nl, nl)])
        pltpu.sync_copy(acc_d.at[pl.ds(sid * nd, nd)],
                        out_d.at[core, pl.ds(sid * nd, nd)])
        if with_counts:
            pltpu.sync_copy(acc_cl.at[pl.ds(sid * nl, nl)],
                            out_cl.at[core, pl.ds(sid * nl, nl)])
            pltpu.sync_copy(acc_cd.at[pl.ds(sid * nd, nd)],
                            out_cd.at[core, pl.ds(sid * nd, nd)])

    return pl.kernel(body, out_type=out_type, mesh=_mesh,
                     scratch_types=scratch, compiler_params=_SC_PARAMS)


def _make_agg_pat_small(n_edges_pad):
    out_type = jax.ShapeDtypeStruct((2, PAT_PAD, DC), f32)
    scratch = [
        pltpu.VMEM((2, CHUNK), i32),
        pltpu.VMEM((2, CHUNK), i32),
        pltpu.VMEM((2 * CHUNK, DC), f32),
        pltpu.VMEM_SHARED((PAT_PAD, DC), f32),
        pltpu.SemaphoreType.DMA,
        pltpu.SemaphoreType.DMA,
    ]
    ept = n_edges_pad // 32

    def body(t, idx_g, idx_s, zp, out, gi, si, rows, acc, sem_g, sem_s):
        core = lax.axis_index("c")
        sid = lax.axis_index("s")
        tile = core * 16 + sid
        npr = PAT_PAD // 16
        pltpu.sync_copy(zp.at[pl.ds(sid * npr, npr)],
                        acc.at[pl.ds(sid * npr, npr)])
        plsc.subcore_barrier()
        _edge_phase(idx_g, idx_s, tile * ept, ept // CHUNK,
                    gi, si, rows, acc, t, sem_g, sem_s, zp, CHUNK)
        plsc.subcore_barrier()
        pltpu.sync_copy(acc.at[pl.ds(sid * npr, npr)],
                        out.at[core, pl.ds(sid * npr, npr)])

    return pl.kernel(body, out_type=out_type, mesh=_mesh,
                     scratch_types=scratch, compiler_params=_SC_PARAMS)


def _make_agg_pat(n_edges_pad):
    chunk = 2048
    out_type = jax.ShapeDtypeStruct((4, PAT_PAD, DB), f32)
    scratch = [
        pltpu.VMEM((2, chunk), i32),
        pltpu.VMEM((2, chunk), i32),
        pltpu.VMEM((2 * chunk, DB), f32),
        pltpu.VMEM_SHARED((PAT_PAD, DB), f32),
        pltpu.SemaphoreType.DMA,
        pltpu.SemaphoreType.DMA,
    ]
    eps = n_edges_pad // 16
    n_chunks = eps // chunk

    def body(t0, t1, t2, t3, idx_g, idx_s, zp, out, gi, si, rows, acc,
             sem_g, sem_s):
        core = lax.axis_index("c")
        sid = lax.axis_index("s")
        npr = PAT_PAD // 16
        row_base = sid * eps
        for q_local, (ta, tb) in enumerate(((t0, t2), (t1, t3))):
            pltpu.sync_copy(zp.at[pl.ds(sid * npr, npr)],
                            acc.at[pl.ds(sid * npr, npr)])
            plsc.subcore_barrier()

            @pl.when(core == 0)
            def _():
                _edge_phase(idx_g, idx_s, row_base, n_chunks, gi, si, rows,
                            acc, ta, sem_g, sem_s, zp, chunk)

            @pl.when(core == 1)
            def _():
                _edge_phase(idx_g, idx_s, row_base, n_chunks, gi, si, rows,
                            acc, tb, sem_g, sem_s, zp, chunk)

            plsc.subcore_barrier()
            pltpu.sync_copy(acc.at[pl.ds(sid * npr, npr)],
                            out.at[2 * core + q_local, pl.ds(sid * npr, npr)])

    return pl.kernel(body, out_type=out_type, mesh=_mesh,
                     scratch_types=scratch, compiler_params=_SC_PARAMS)


PB = 2000
NPB = NP_N // PB


def _full(shape):
    return pl.BlockSpec(shape, lambda i: (0,) * len(shape))


def _rows(w):
    return pl.BlockSpec((PB, w), lambda i: (i, 0))


def _dot(a, b):
    return jnp.dot(a, b, preferred_element_type=f32)


def _node_dense_l1(a0, a1, c0, c1, x_raw, w_pat, b_pat, w_own, b_own,
                   wl, wr, bias, s, t, n, k_raw):
    def body(a0r, a1r, c0r, c1r, xr, wpr, bpr, wor, bor, wlr, wrr, br, sr, tr,
             o_ref, cnt_ref):
        cnt = c0r[:, 0:1] + c1r[:, 0:1]
        cnt_ref[...] = cnt
        mean_raw = (a0r[...] + a1r[...]) / jnp.maximum(cnt, 1.0)
        xm = _dot(mean_raw, wpr[...]) + bpr[...]
        x_own = bor[...]
        for j in range(k_raw):
            x_own = x_own + xr[:, j:j + 1] * wor[j:j + 1, :]
        o = _dot(xm, wlr[...]) + _dot(x_own, wrr[...]) + br[...]
        o_ref[...] = jnp.maximum(o * sr[...] + tr[...], 0.0)

    res = pl.pallas_call(
        body, grid=(1,),
        in_specs=[_full((n, 16)), _full((n, 16)), _full((n, DC)),
                  _full((n, DC)), _full((n, k_raw)), _full((16, H)),
                  _full((1, H)), _full((k_raw, H)), _full((1, H)),
                  _full((H, H)), _full((H, H)), _full((1, H)),
                  _full((1, H)), _full((1, H))],
        out_specs=[_full((n, H)), _full((n, 1))],
        out_shape=[jax.ShapeDtypeStruct((n, H), f32),
                   jax.ShapeDtypeStruct((n, 1), f32)],
    )(a0, a1, c0, c1, x_raw, w_pat, b_pat, w_own, b_own, wl, wr, bias, s, t)
    return res


def _node_dense_l2(a0, a1, cnt, x, wl, wr, bias, s, t, n):
    def body(a0r, a1r, cr, xr, wlr, wrr, br, sr, tr, o_ref):
        mean = (a0r[...] + a1r[...]) / jnp.maximum(cr[...], 1.0)
        o = _dot(mean, wlr[...]) + _dot(xr[...], wrr[...]) + br[...]
        o_ref[...] = jnp.maximum(o * sr[...] + tr[...], 0.0)

    return pl.pallas_call(
        body, grid=(1,),
        in_specs=[_full((n, H)), _full((n, H)), _full((n, 1)), _full((n, H)),
                  _full((H, H)), _full((H, H)), _full((1, H)), _full((1, H)),
                  _full((1, H))],
        out_specs=_full((n, H)),
        out_shape=jax.ShapeDtypeStruct((n, H), f32),
    )(a0, a1, cnt, x, wl, wr, bias, s, t)


def _pat_dense_l1(b0, b1, c0, c1, x_raw, w_pat, b_pat, w_lab, b_lab,
                  w_dis, b_dis, wl_l, wl_d, wr, bias, s, t):
    def body(b0r, b1r, c0r, c1r, xr, wpr, bpr, wlbr, blbr, wdr, bdr,
             wllr, wldr, wrr, br, sr, tr, o_ref, cnt_ref):
        cl = b0r[:, 1:2] + b1r[:, 1:2]
        cd = c0r[:, 2:3] + c1r[:, 2:3]
        cnt_ref[...] = jnp.concatenate([cl, cd], axis=1)
        mean_u = (b0r[:, 0:1] + b1r[:, 0:1]) / jnp.maximum(cl, 1.0)
        xl_mean = mean_u * wlbr[...] + blbr[...]
        cdm = jnp.maximum(cd, 1.0)
        d0 = (c0r[:, 0:1] + c1r[:, 0:1]) / cdm
        d1 = (c0r[:, 1:2] + c1r[:, 1:2]) / cdm
        xd_mean = d0 * wdr[0:1, :] + d1 * wdr[1:2, :] + bdr[...]
        x_own = _dot(xr[...], wpr[...]) + bpr[...]
        o = (_dot(xl_mean, wllr[...]) + _dot(xd_mean, wldr[...])
             + _dot(x_own, wrr[...]) + br[...])
        o_ref[...] = jnp.maximum(o * sr[...] + tr[...], 0.0)

    return pl.pallas_call(
        body, grid=(NPB,),
        in_specs=[_rows(DC)] * 4 + [_rows(16)]
        + [_full((16, H)), _full((1, H)), _full((1, H)), _full((1, H)),
           _full((2, H)), _full((1, H)), _full((H, H)), _full((H, H)),
           _full((H, H)), _full((1, H)), _full((1, H)), _full((1, H))],
        out_specs=[_rows(H), _rows(2)],
        out_shape=[jax.ShapeDtypeStruct((NP_N, H), f32),
                   jax.ShapeDtypeStruct((NP_N, 2), f32)],
    )(b0, b1, c0, c1, x_raw, w_pat, b_pat, w_lab, b_lab, w_dis, b_dis,
      wl_l, wl_d, wr, bias, s, t)


def _pat_dense_l2(bq, cq, cnt, xp, wl_l, wl_d, wr, bias, s, t):
    wl_ls = [wl_l[16 * q:16 * (q + 1)] for q in range(4)]
    wl_ds = [wl_d[16 * q:16 * (q + 1)] for q in range(4)]

    def body(b0r, b1r, b2r, b3r, c0r, c1r, c2r, c3r, cr, xr,
             wll0, wll1, wll2, wll3, wld0, wld1, wld2, wld3,
             wrr, br, sr, tr, o_ref):
        cl = jnp.maximum(cr[:, 0:1], 1.0)
        cd = jnp.maximum(cr[:, 1:2], 1.0)
        o = _dot(xr[...], wrr[...]) + br[...]
        for qr, wq in zip((b0r, b1r, b2r, b3r), (wll0, wll1, wll2, wll3)):
            o = o + _dot(qr[...] / cl, wq[...])
        for qr, wq in zip((c0r, c1r, c2r, c3r), (wld0, wld1, wld2, wld3)):
            o = o + _dot(qr[...] / cd, wq[...])
        o_ref[...] = jnp.maximum(o * sr[...] + tr[...], 0.0)

    return pl.pallas_call(
        body, grid=(NPB,),
        in_specs=[_rows(DB)] * 8 + [_rows(2), _rows(H)]
        + [_full((16, H))] * 8
        + [_full((H, H)), _full((1, H)), _full((1, H)), _full((1, H))],
        out_specs=_rows(H),
        out_shape=jax.ShapeDtypeStruct((NP_N, H), f32),
    )(*bq, *cq, cnt, xp, *wl_ls, *wl_ds, wr, bias, s, t)


def _head(xp, w1, b1, w2, b2):
    def body(x_ref, w1_ref, b1_ref, w2_ref, b2_ref, o_ref):
        h = jnp.maximum(_dot(x_ref[...], w1_ref[...]) + b1_ref[...], 0.0)
        logits = _dot(h, w2_ref[...]) + b2_ref[...]
        m = jnp.max(logits, axis=1, keepdims=True)
        lse = jnp.log(jnp.sum(jnp.exp(logits - m), axis=1, keepdims=True)) + m
        o_ref[...] = logits - lse

    return pl.pallas_call(
        body, grid=(NPB,),
        in_specs=[_rows(H), _full((H, H // 2)), _full((1, H // 2)),
                  _full((H // 2, OUT)), _full((1, OUT))],
        out_specs=_rows(OUT),
        out_shape=jax.ShapeDtypeStruct((NP_N, OUT), f32),
    )(xp, w1, b1, w2, b2)


_agg_ld1 = _make_agg_ld(16, True, 1024)
_agg_ld2 = _make_agg_ld(H, False, 512)
_agg_pl1 = _make_agg_pat_small(EPL_PAD)
_agg_pd1 = _make_agg_pat_small(EPD_PAD)
_agg_pl2 = _make_agg_pat(EPL_PAD)
_agg_pd2 = _make_agg_pat(EPD_PAD)

_BN_INV = 1.0 / math.sqrt(1.0 + 1e-5)


def _pad_idx(idx, n_pad, fill):
    return jnp.concatenate(
        [idx, jnp.full((n_pad - idx.shape[0],), fill, i32)])


def kernel(x_patient, x_lab, x_disease, src_pl, dst_pl, src_pd, dst_pd,
           W_pat, b_pat, W_lab, b_lab, W_dis, b_dis,
           c1_pl_Wl, c1_pl_bl, c1_pl_Wr, c1_pd_Wl, c1_pd_bl, c1_pd_Wr,
           c1_lp_Wl, c1_lp_bl, c1_lp_Wr, c1_dp_Wl, c1_dp_bl, c1_dp_Wr,
           c2_pl_Wl, c2_pl_bl, c2_pl_Wr, c2_pd_Wl, c2_pd_bl, c2_pd_Wr,
           c2_lp_Wl, c2_lp_bl, c2_lp_Wr, c2_dp_Wl, c2_dp_bl, c2_dp_Wr,
           bn1_p_g, bn1_p_b, bn1_l_g, bn1_l_b, bn1_d_g, bn1_d_b,
           bn2_p_g, bn2_p_b, bn2_l_g, bn2_l_b, bn2_d_g, bn2_d_b,
           W_fc1, b_fc1, W_fc2, b_fc2):
    row = lambda v: v.reshape(1, -1)

    spl_gA = _pad_idx(src_pl, EPL_PAD, 0)
    dpl_sA = _pad_idx(dst_pl, EPL_PAD, NL_N)
    dpl_gB = _pad_idx(dst_pl, EPL_PAD, 0)
    spl_sB = _pad_idx(src_pl, EPL_PAD, NP_N)
    spd_gA = _pad_idx(src_pd, EPD_PAD, 0)
    dpd_sA = _pad_idx(dst_pd, EPD_PAD, ND_N)
    dpd_gC = _pad_idx(dst_pd, EPD_PAD, 0)
    spd_sC = _pad_idx(src_pd, EPD_PAD, NP_N)

    z16l = jnp.zeros((LAB_PAD, 16), f32)
    z16d = jnp.zeros((DIS_PAD, 16), f32)
    z64l = jnp.zeros((LAB_PAD, H), f32)
    z64d = jnp.zeros((DIS_PAD, H), f32)
    zc8l = jnp.zeros((LAB_PAD, DC), f32)
    zc8d = jnp.zeros((DIS_PAD, DC), f32)
    zp16 = jnp.zeros((PAT_PAD, DB), f32)
    zp8 = jnp.zeros((PAT_PAD, DC), f32)
    ones_rows = jnp.ones((CHUNK, DC), f32)

    tl8 = jnp.concatenate(
        [x_lab, jnp.ones((NL_N, 1), f32), jnp.zeros((NL_N, DC - 2), f32)],
        axis=1)
    td8 = jnp.concatenate(
        [x_disease, jnp.ones((ND_N, 1), f32), jnp.zeros((ND_N, DC - 3), f32)],
        axis=1)

    out_l, out_d, oc_l, oc_d = _agg_ld1(
        x_patient, spl_gA, dpl_sA, spd_gA, dpd_sA, z16l, z16d, zc8l, zc8d,
        ones_rows)
    out_b = _agg_pl1(tl8, dpl_gB, spl_sB, zp8)
    out_c = _agg_pd1(td8, dpd_gC, spd_sC, zp8)

    s_l1, s_d1, s_p1 = (row(bn1_l_g) * _BN_INV, row(bn1_d_g) * _BN_INV,
                        row(bn1_p_g) * _BN_INV)
    x_l, cnt_l = _node_dense_l1(
        out_l[0, :NL_N], out_l[1, :NL_N], oc_l[0, :NL_N], oc_l[1, :NL_N],
        x_lab, W_pat, row(b_pat), W_lab, row(b_lab), c1_pl_Wl, c1_pl_Wr,
        row(c1_pl_bl), s_l1, row(bn1_l_b), NL_N, 1)
    x_d, cnt_d = _node_dense_l1(
        out_d[0, :ND_N], out_d[1, :ND_N], oc_d[0, :ND_N], oc_d[1, :ND_N],
        x_disease, W_pat, row(b_pat), W_dis, row(b_dis), c1_pd_Wl, c1_pd_Wr,
        row(c1_pd_bl), s_d1, row(bn1_d_b), ND_N, 2)
    x_p, cnt_p = _pat_dense_l1(
        out_b[0, :NP_N], out_b[1, :NP_N], out_c[0, :NP_N], out_c[1, :NP_N],
        x_patient, W_pat, row(b_pat), row(W_lab[0]), row(b_lab), W_dis,
        row(b_dis), c1_lp_Wl, c1_dp_Wl, c1_lp_Wr + c1_dp_Wr,
        row(c1_lp_bl + c1_dp_bl), s_p1, row(bn1_p_b))

    tls = [x_l[:, 16 * q:16 * (q + 1)] for q in range(4)]
    tds = [x_d[:, 16 * q:16 * (q + 1)] for q in range(4)]

    out_l2, out_d2 = _agg_ld2(x_p, spl_gA, dpl_sA, spd_gA, dpd_sA, z64l, z64d)
    out_b2 = _agg_pl2(*tls, dpl_gB, spl_sB, zp16)
    out_c2 = _agg_pd2(*tds, dpd_gC, spd_sC, zp16)

    s_l2, s_d2, s_p2 = (row(bn2_l_g) * _BN_INV, row(bn2_d_g) * _BN_INV,
                        row(bn2_p_g) * _BN_INV)
    x_l = _node_dense_l2(out_l2[0, :NL_N], out_l2[1, :NL_N], cnt_l, x_l,
                         c2_pl_Wl, c2_pl_Wr, row(c2_pl_bl), s_l2,
                         row(bn2_l_b), NL_N)
    x_d = _node_dense_l2(out_d2[0, :ND_N], out_d2[1, :ND_N], cnt_d, x_d,
                         c2_pd_Wl, c2_pd_Wr, row(c2_pd_bl), s_d2,
                         row(bn2_d_b), ND_N)
    bq = [out_b2[q, :NP_N] for q in range(4)]
    cq = [out_c2[q, :NP_N] for q in range(4)]
    x_p = _pat_dense_l2(bq, cq, cnt_p, x_p, c2_lp_Wl, c2_dp_Wl,
                        c2_lp_Wr + c2_dp_Wr, row(c2_lp_bl + c2_dp_bl),
                        s_p2, row(bn2_p_b))

    return _head(x_p, W_fc1, row(b_fc1), W_fc2, row(b_fc2))


# speedup vs baseline: 5.6840x; 1.0147x over previous
import functools
import math

import jax
import jax.numpy as jnp
from jax import lax
from jax.experimental import pallas as pl
from jax.experimental.pallas import tpu as pltpu
from jax.experimental.pallas import tpu_sc as plsc

NP_N, NL_N, ND_N = 50000, 5000, 500
EPL_PAD, EPD_PAD = 819200, 229376
H = 64
OUT = 50
LAB_PAD, DIS_PAD, PAT_PAD = 5120, 512, 50048
DB = 16
DC = 8
CHUNK = 1024
KROWS = CHUNK // 128

_mesh = plsc.VectorSubcoreMesh(core_axis_name="c", subcore_axis_name="s")
_SC_PARAMS = pltpu.CompilerParams(use_tc_tiling_on_sc=False)

f32 = jnp.float32
i32 = jnp.int32


def _edge_phase(idx_g_hbm, idx_s_hbm, row_base, n_chunks, gi, si, rows, acc,
                table_ref, sem_g, sem_s, dummy, chunk):
    def load_idx(t, half):
        eb = row_base + t * chunk
        pltpu.sync_copy(idx_g_hbm.at[pl.ds(eb, chunk)], gi.at[half])
        pltpu.sync_copy(idx_s_hbm.at[pl.ds(eb, chunk)], si.at[half])

    def fire_gathers(half):
        pltpu.async_copy(table_ref.at[gi.at[half]],
                         rows.at[pl.ds(half * chunk, chunk)], sem_g)

    def fire_scatters(half):
        pltpu.async_copy(rows.at[pl.ds(half * chunk, chunk)],
                         acc.at[si.at[half]], sem_s, add=True)

    def drain(sem):
        pltpu.make_async_copy(dummy.at[pl.ds(0, chunk)],
                              rows.at[pl.ds(0, chunk)], sem).wait()

    load_idx(0, 0)
    fire_gathers(0)

    def body(t, carry):
        for p in (0, 1):
            @pl.when(lax.rem(t, 2) == p)
            def _():
                o = 1 - p
                load_idx(t + 1, o)

                @pl.when(t >= 1)
                def _():
                    drain(sem_s)

                fire_gathers(o)
                drain(sem_g)
                fire_scatters(p)
        return carry

    lax.fori_loop(0, n_chunks - 1, body, 0)
    drain(sem_g)
    fire_scatters((n_chunks - 1) % 2)
    for _ in range(min(2, n_chunks)):
        drain(sem_s)


def _count_phase(idx_s_hbm, edge_base, n_chunks, si, ones_rows, acc, sem_s):
    def body(t, carry):
        eb = edge_base + t * CHUNK
        pltpu.sync_copy(idx_s_hbm.at[pl.ds(eb, CHUNK)], si.at[0])
        pltpu.async_copy(ones_rows, acc.at[si.at[0]], sem_s,
                         add=True).wait()
        return carry

    lax.fori_loop(0, n_chunks, body, 0)


def _make_agg_ld(width, with_counts, chunk):
    out_type = [
        jax.ShapeDtypeStruct((2, LAB_PAD, width), f32),
        jax.ShapeDtypeStruct((2, DIS_PAD, width), f32),
    ]
    scratch = [
        pltpu.VMEM((2, chunk), i32),
        pltpu.VMEM((2, chunk), i32),
        pltpu.VMEM((2 * chunk, width), f32),
        pltpu.VMEM_SHARED((LAB_PAD, width), f32),
        pltpu.VMEM_SHARED((DIS_PAD, width), f32),
        pltpu.SemaphoreType.DMA,
        pltpu.SemaphoreType.DMA,
    ]
    if with_counts:
        out_type += [
            jax.ShapeDtypeStruct((2, LAB_PAD, DC), f32),
            jax.ShapeDtypeStruct((2, DIS_PAD, DC), f32),
        ]
        scratch += [
            pltpu.VMEM((CHUNK, DC), f32),
            pltpu.VMEM_SHARED((LAB_PAD, DC), f32),
            pltpu.VMEM_SHARED((DIS_PAD, DC), f32),
        ]

    ept_pl = EPL_PAD // 32
    ept_pd = EPD_PAD // 32

    def body(*refs):
        if with_counts:
            (tp, spl_g, dpl_s, spd_g, dpd_s, zl, zd, zcl, zcd, ones_hbm,
             out_l, out_d, out_cl, out_cd,
             gi, si, rows, acc_l, acc_d, sem_g, sem_s,
             ones_rows, acc_cl, acc_cd) = refs
        else:
            (tp, spl_g, dpl_s, spd_g, dpd_s, zl, zd, out_l, out_d,
             gi, si, rows, acc_l, acc_d, sem_g, sem_s) = refs
        core = lax.axis_index("c")
        sid = lax.axis_index("s")
        tile = core * 16 + sid
        nl = LAB_PAD // 16
        nd = DIS_PAD // 16
        pltpu.sync_copy(zl.at[pl.ds(sid * nl, nl)],
                        acc_l.at[pl.ds(sid * nl, nl)])
        pltpu.sync_copy(zd.at[pl.ds(sid * nd, nd)],
                        acc_d.at[pl.ds(sid * nd, nd)])
        if with_counts:
            pltpu.sync_copy(zcl.at[pl.ds(sid * nl, nl)],
                            acc_cl.at[pl.ds(sid * nl, nl)])
            pltpu.sync_copy(zcd.at[pl.ds(sid * nd, nd)],
                            acc_cd.at[pl.ds(sid * nd, nd)])
            pltpu.sync_copy(ones_hbm, ones_rows)
        plsc.subcore_barrier()
        _edge_phase(spl_g, dpl_s, tile * ept_pl, ept_pl // chunk,
                    gi, si, rows, acc_l, tp, sem_g, sem_s, zl, chunk)
        _edge_phase(spd_g, dpd_s, tile * ept_pd, ept_pd // chunk,
                    gi, si, rows, acc_d, tp, sem_g, sem_s, zl, chunk)
        if with_counts:
            _count_phase(dpl_s, tile * ept_pl, ept_pl // CHUNK,
                         si, ones_rows, acc_cl, sem_s)
            _count_phase(dpd_s, tile * ept_pd, ept_pd // CHUNK,
                         si, ones_rows, acc_cd, sem_s)
        plsc.subcore_barrier()
        pltpu.sync_copy(acc_l.at[pl.ds(sid * nl, nl)],
                        out_l.at[core, pl.ds(sid * nl, nl)])
        pltpu.sync_copy(acc_d.at[pl.ds(sid * nd, nd)],
                        out_d.at[core, pl.ds(sid * nd, nd)])
        if with_counts:
            pltpu.sync_copy(acc_cl.at[pl.ds(sid * nl, nl)],
                            out_cl.at[core, pl.ds(sid * nl, nl)])
            pltpu.sync_copy(acc_cd.at[pl.ds(sid * nd, nd)],
                            out_cd.at[core, pl.ds(sid * nd, nd)])

    return pl.kernel(body, out_type=out_type, mesh=_mesh,
                     scratch_types=scratch, compiler_params=_SC_PARAMS)


def _make_agg_pat_small(n_edges_pad):
    out_type = jax.ShapeDtypeStruct((2, PAT_PAD, DC), f32)
    scratch = [
        pltpu.VMEM((2, CHUNK), i32),
        pltpu.VMEM((2, CHUNK), i32),
        pltpu.VMEM((2 * CHUNK, DC), f32),
        pltpu.VMEM_SHARED((PAT_PAD, DC), f32),
        pltpu.SemaphoreType.DMA,
        pltpu.SemaphoreType.DMA,
    ]
    ept = n_edges_pad // 32

    def body(t, idx_g, idx_s, zp, out, gi, si, rows, acc, sem_g, sem_s):
        core = lax.axis_index("c")
        sid = lax.axis_index("s")
        tile = core * 16 + sid
        npr = PAT_PAD // 16
        pltpu.sync_copy(zp.at[pl.ds(sid * npr, npr)],
                        acc.at[pl.ds(sid * npr, npr)])
        plsc.subcore_barrier()
        _edge_phase(idx_g, idx_s, tile * ept, ept // CHUNK,
                    gi, si, rows, acc, t, sem_g, sem_s, zp, CHUNK)
        plsc.subcore_barrier()
        pltpu.sync_copy(acc.at[pl.ds(sid * npr, npr)],
                        out.at[core, pl.ds(sid * npr, npr)])

    return pl.kernel(body, out_type=out_type, mesh=_mesh,
                     scratch_types=scratch, compiler_params=_SC_PARAMS)


def _make_agg_pat(n_edges_pad):
    chunk = 2048
    out_type = jax.ShapeDtypeStruct((4, PAT_PAD, DB), f32)
    scratch = [
        pltpu.VMEM((2, chunk), i32),
        pltpu.VMEM((2, chunk), i32),
        pltpu.VMEM((2 * chunk, DB), f32),
        pltpu.VMEM_SHARED((PAT_PAD, DB), f32),
        pltpu.SemaphoreType.DMA,
        pltpu.SemaphoreType.DMA,
    ]
    eps = n_edges_pad // 16
    n_chunks = eps // chunk

    def body(t0, t1, t2, t3, idx_g, idx_s, zp, out, gi, si, rows, acc,
             sem_g, sem_s):
        core = lax.axis_index("c")
        sid = lax.axis_index("s")
        npr = PAT_PAD // 16
        row_base = sid * eps
        for q_local, (ta, tb) in enumerate(((t0, t2), (t1, t3))):
            pltpu.sync_copy(zp.at[pl.ds(sid * npr, npr)],
                            acc.at[pl.ds(sid * npr, npr)])
            plsc.subcore_barrier()

            @pl.when(core == 0)
            def _():
                _edge_phase(idx_g, idx_s, row_base, n_chunks, gi, si, rows,
                            acc, ta, sem_g, sem_s, zp, chunk)

            @pl.when(core == 1)
            def _():
                _edge_phase(idx_g, idx_s, row_base, n_chunks, gi, si, rows,
                            acc, tb, sem_g, sem_s, zp, chunk)

            plsc.subcore_barrier()
            pltpu.sync_copy(acc.at[pl.ds(sid * npr, npr)],
                            out.at[2 * core + q_local, pl.ds(sid * npr, npr)])

    return pl.kernel(body, out_type=out_type, mesh=_mesh,
                     scratch_types=scratch, compiler_params=_SC_PARAMS)


PB = 2000
NPB = NP_N // PB


def _full(shape):
    return pl.BlockSpec(shape, lambda i: (0,) * len(shape))


def _rows(w):
    return pl.BlockSpec((PB, w), lambda i: (i, 0))


def _dot(a, b):
    return jnp.dot(a, b, preferred_element_type=f32)


def _node_dense_l1(a0, a1, c0, c1, x_raw, w_pat, b_pat, w_own, b_own,
                   wl, wr, bias, s, t, n, k_raw):
    def body(a0r, a1r, c0r, c1r, xr, wpr, bpr, wor, bor, wlr, wrr, br, sr, tr,
             o_ref, cnt_ref):
        cnt = c0r[:, 0:1] + c1r[:, 0:1]
        cnt_ref[...] = cnt
        mean_raw = (a0r[...] + a1r[...]) / jnp.maximum(cnt, 1.0)
        xm = _dot(mean_raw, wpr[...]) + bpr[...]
        x_own = bor[...]
        for j in range(k_raw):
            x_own = x_own + xr[:, j:j + 1] * wor[j:j + 1, :]
        o = _dot(xm, wlr[...]) + _dot(x_own, wrr[...]) + br[...]
        o_ref[...] = jnp.maximum(o * sr[...] + tr[...], 0.0)

    res = pl.pallas_call(
        body, grid=(1,),
        in_specs=[_full((n, 16)), _full((n, 16)), _full((n, DC)),
                  _full((n, DC)), _full((n, k_raw)), _full((16, H)),
                  _full((1, H)), _full((k_raw, H)), _full((1, H)),
                  _full((H, H)), _full((H, H)), _full((1, H)),
                  _full((1, H)), _full((1, H))],
        out_specs=[_full((n, H)), _full((n, 1))],
        out_shape=[jax.ShapeDtypeStruct((n, H), f32),
                   jax.ShapeDtypeStruct((n, 1), f32)],
    )(a0, a1, c0, c1, x_raw, w_pat, b_pat, w_own, b_own, wl, wr, bias, s, t)
    return res


def _node_dense_l2(a0, a1, cnt, x, wl, wr, bias, s, t, n):
    def body(a0r, a1r, cr, xr, wlr, wrr, br, sr, tr, o_ref):
        mean = (a0r[...] + a1r[...]) / jnp.maximum(cr[...], 1.0)
        o = _dot(mean, wlr[...]) + _dot(xr[...], wrr[...]) + br[...]
        o_ref[...] = jnp.maximum(o * sr[...] + tr[...], 0.0)

    return pl.pallas_call(
        body, grid=(1,),
        in_specs=[_full((n, H)), _full((n, H)), _full((n, 1)), _full((n, H)),
                  _full((H, H)), _full((H, H)), _full((1, H)), _full((1, H)),
                  _full((1, H))],
        out_specs=_full((n, H)),
        out_shape=jax.ShapeDtypeStruct((n, H), f32),
    )(a0, a1, cnt, x, wl, wr, bias, s, t)


def _pat_dense_l1(b0, b1, c0, c1, x_raw, w_pat, b_pat, w_lab, b_lab,
                  w_dis, b_dis, wl_l, wl_d, wr, bias, s, t):
    def body(b0r, b1r, c0r, c1r, xr, wpr, bpr, wlbr, blbr, wdr, bdr,
             wllr, wldr, wrr, br, sr, tr, o_ref, cnt_ref):
        cl = b0r[:, 1:2] + b1r[:, 1:2]
        cd = c0r[:, 2:3] + c1r[:, 2:3]
        cnt_ref[...] = jnp.concatenate([cl, cd], axis=1)
        mean_u = (b0r[:, 0:1] + b1r[:, 0:1]) / jnp.maximum(cl, 1.0)
        xl_mean = mean_u * wlbr[...] + blbr[...]
        cdm = jnp.maximum(cd, 1.0)
        d0 = (c0r[:, 0:1] + c1r[:, 0:1]) / cdm
        d1 = (c0r[:, 1:2] + c1r[:, 1:2]) / cdm
        xd_mean = d0 * wdr[0:1, :] + d1 * wdr[1:2, :] + bdr[...]
        x_own = _dot(xr[...], wpr[...]) + bpr[...]
        o = (_dot(xl_mean, wllr[...]) + _dot(xd_mean, wldr[...])
             + _dot(x_own, wrr[...]) + br[...])
        o_ref[...] = jnp.maximum(o * sr[...] + tr[...], 0.0)

    return pl.pallas_call(
        body, grid=(NPB,),
        in_specs=[_rows(DC)] * 4 + [_rows(16)]
        + [_full((16, H)), _full((1, H)), _full((1, H)), _full((1, H)),
           _full((2, H)), _full((1, H)), _full((H, H)), _full((H, H)),
           _full((H, H)), _full((1, H)), _full((1, H)), _full((1, H))],
        out_specs=[_rows(H), _rows(2)],
        out_shape=[jax.ShapeDtypeStruct((NP_N, H), f32),
                   jax.ShapeDtypeStruct((NP_N, 2), f32)],
    )(b0, b1, c0, c1, x_raw, w_pat, b_pat, w_lab, b_lab, w_dis, b_dis,
      wl_l, wl_d, wr, bias, s, t)


def _pat_dense_l2_head(bq, cq, cnt, xp, wl_l, wl_d, wr, bias, s, t,
                       w1, b1, w2, b2):
    wl_ls = [wl_l[16 * q:16 * (q + 1)] for q in range(4)]
    wl_ds = [wl_d[16 * q:16 * (q + 1)] for q in range(4)]

    def body(b0r, b1r, b2r, b3r, c0r, c1r, c2r, c3r, cr, xr,
             wll0, wll1, wll2, wll3, wld0, wld1, wld2, wld3,
             wrr, br, sr, tr, w1r, b1r_, w2r, b2r_, o_ref):
        cl = jnp.maximum(cr[:, 0:1], 1.0)
        cd = jnp.maximum(cr[:, 1:2], 1.0)
        o = _dot(xr[...], wrr[...]) + br[...]
        for qr, wq in zip((b0r, b1r, b2r, b3r), (wll0, wll1, wll2, wll3)):
            o = o + _dot(qr[...] / cl, wq[...])
        for qr, wq in zip((c0r, c1r, c2r, c3r), (wld0, wld1, wld2, wld3)):
            o = o + _dot(qr[...] / cd, wq[...])
        x2 = jnp.maximum(o * sr[...] + tr[...], 0.0)
        h = jnp.maximum(_dot(x2, w1r[...]) + b1r_[...], 0.0)
        logits = _dot(h, w2r[...]) + b2r_[...]
        m = jnp.max(logits, axis=1, keepdims=True)
        lse = jnp.log(jnp.sum(jnp.exp(logits - m), axis=1, keepdims=True)) + m
        o_ref[...] = logits - lse

    return pl.pallas_call(
        body, grid=(NPB,),
        in_specs=[_rows(DB)] * 8 + [_rows(2), _rows(H)]
        + [_full((16, H))] * 8
        + [_full((H, H)), _full((1, H)), _full((1, H)), _full((1, H)),
           _full((H, H // 2)), _full((1, H // 2)), _full((H // 2, OUT)),
           _full((1, OUT))],
        out_specs=_rows(OUT),
        out_shape=jax.ShapeDtypeStruct((NP_N, OUT), f32),
    )(*bq, *cq, cnt, xp, *wl_ls, *wl_ds, wr, bias, s, t, w1, b1, w2, b2)


_agg_ld1 = _make_agg_ld(16, True, 1024)
_agg_pl1 = _make_agg_pat_small(EPL_PAD)
_agg_pd1 = _make_agg_pat_small(EPD_PAD)
_agg_pl2 = _make_agg_pat(EPL_PAD)
_agg_pd2 = _make_agg_pat(EPD_PAD)

_BN_INV = 1.0 / math.sqrt(1.0 + 1e-5)


def _pad_idx(idx, n_pad, fill):
    return jnp.concatenate(
        [idx, jnp.full((n_pad - idx.shape[0],), fill, i32)])


def kernel(x_patient, x_lab, x_disease, src_pl, dst_pl, src_pd, dst_pd,
           W_pat, b_pat, W_lab, b_lab, W_dis, b_dis,
           c1_pl_Wl, c1_pl_bl, c1_pl_Wr, c1_pd_Wl, c1_pd_bl, c1_pd_Wr,
           c1_lp_Wl, c1_lp_bl, c1_lp_Wr, c1_dp_Wl, c1_dp_bl, c1_dp_Wr,
           c2_pl_Wl, c2_pl_bl, c2_pl_Wr, c2_pd_Wl, c2_pd_bl, c2_pd_Wr,
           c2_lp_Wl, c2_lp_bl, c2_lp_Wr, c2_dp_Wl, c2_dp_bl, c2_dp_Wr,
           bn1_p_g, bn1_p_b, bn1_l_g, bn1_l_b, bn1_d_g, bn1_d_b,
           bn2_p_g, bn2_p_b, bn2_l_g, bn2_l_b, bn2_d_g, bn2_d_b,
           W_fc1, b_fc1, W_fc2, b_fc2):
    row = lambda v: v.reshape(1, -1)

    spl_gA = _pad_idx(src_pl, EPL_PAD, 0)
    dpl_sA = _pad_idx(dst_pl, EPL_PAD, NL_N)
    dpl_gB = _pad_idx(dst_pl, EPL_PAD, 0)
    spl_sB = _pad_idx(src_pl, EPL_PAD, NP_N)
    spd_gA = _pad_idx(src_pd, EPD_PAD, 0)
    dpd_sA = _pad_idx(dst_pd, EPD_PAD, ND_N)
    dpd_gC = _pad_idx(dst_pd, EPD_PAD, 0)
    spd_sC = _pad_idx(src_pd, EPD_PAD, NP_N)

    z16l = jnp.zeros((LAB_PAD, 16), f32)
    z16d = jnp.zeros((DIS_PAD, 16), f32)
    zc8l = jnp.zeros((LAB_PAD, DC), f32)
    zc8d = jnp.zeros((DIS_PAD, DC), f32)
    zp16 = jnp.zeros((PAT_PAD, DB), f32)
    zp8 = jnp.zeros((PAT_PAD, DC), f32)
    ones_rows = jnp.ones((CHUNK, DC), f32)

    tl8 = jnp.concatenate(
        [x_lab, jnp.ones((NL_N, 1), f32), jnp.zeros((NL_N, DC - 2), f32)],
        axis=1)
    td8 = jnp.concatenate(
        [x_disease, jnp.ones((ND_N, 1), f32), jnp.zeros((ND_N, DC - 3), f32)],
        axis=1)

    out_l, out_d, oc_l, oc_d = _agg_ld1(
        x_patient, spl_gA, dpl_sA, spd_gA, dpd_sA, z16l, z16d, zc8l, zc8d,
        ones_rows)
    out_b = _agg_pl1(tl8, dpl_gB, spl_sB, zp8)
    out_c = _agg_pd1(td8, dpd_gC, spd_sC, zp8)

    s_l1, s_d1, s_p1 = (row(bn1_l_g) * _BN_INV, row(bn1_d_g) * _BN_INV,
                        row(bn1_p_g) * _BN_INV)
    x_l, cnt_l = _node_dense_l1(
        out_l[0, :NL_N], out_l[1, :NL_N], oc_l[0, :NL_N], oc_l[1, :NL_N],
        x_lab, W_pat, row(b_pat), W_lab, row(b_lab), c1_pl_Wl, c1_pl_Wr,
        row(c1_pl_bl), s_l1, row(bn1_l_b), NL_N, 1)
    x_d, cnt_d = _node_dense_l1(
        out_d[0, :ND_N], out_d[1, :ND_N], oc_d[0, :ND_N], oc_d[1, :ND_N],
        x_disease, W_pat, row(b_pat), W_dis, row(b_dis), c1_pd_Wl, c1_pd_Wr,
        row(c1_pd_bl), s_d1, row(bn1_d_b), ND_N, 2)
    x_p, cnt_p = _pat_dense_l1(
        out_b[0, :NP_N], out_b[1, :NP_N], out_c[0, :NP_N], out_c[1, :NP_N],
        x_patient, W_pat, row(b_pat), row(W_lab[0]), row(b_lab), W_dis,
        row(b_dis), c1_lp_Wl, c1_dp_Wl, c1_lp_Wr + c1_dp_Wr,
        row(c1_lp_bl + c1_dp_bl), s_p1, row(bn1_p_b))

    tls = [x_l[:, 16 * q:16 * (q + 1)] for q in range(4)]
    tds = [x_d[:, 16 * q:16 * (q + 1)] for q in range(4)]

    out_b2 = _agg_pl2(*tls, dpl_gB, spl_sB, zp16)
    out_c2 = _agg_pd2(*tds, dpd_gC, spd_sC, zp16)

    s_p2 = row(bn2_p_g) * _BN_INV
    bq = [out_b2[q, :NP_N] for q in range(4)]
    cq = [out_c2[q, :NP_N] for q in range(4)]
    return _pat_dense_l2_head(
        bq, cq, cnt_p, x_p, c2_lp_Wl, c2_dp_Wl, c2_lp_Wr + c2_dp_Wr,
        row(c2_lp_bl + c2_dp_bl), s_p2, row(bn2_p_b),
        W_fc1, row(b_fc1), W_fc2, row(b_fc2))
